# baseline (device time: 15533 ns/iter reference)
import jax
import jax.numpy as jnp
from jax import lax
from jax.experimental import pallas as pl
from jax.experimental.pallas import tpu as pltpu

N_DEV = 8
B, SQ, D_MODEL = 2, 128, 512
H_PER, DH = 4, 64
ROWS = B * SQ
ROWS_PER = ROWS // N_DEV
NC = 2
CW = D_MODEL // NC


def kernel(x, Wq, K_ext, V_ext, Wo):
    my = lax.axis_index("i")
    Ks = lax.dynamic_slice_in_dim(K_ext, my * H_PER, H_PER, axis=2)
    Vs = lax.dynamic_slice_in_dim(V_ext, my * H_PER, H_PER, axis=2)
    xf = x.reshape(ROWS, D_MODEL)

    def body(x_ref, wq_ref, k_ref, v_ref, wo_ref, out_ref,
             partial_ref, comm_a, comm_b, gather_ref,
             local_sems, send_a, recv_a, send_b, recv_b):
        my_pos = lax.axis_index("i")

        barrier_sem = pltpu.get_barrier_semaphore()
        for o in range(1, N_DEV):
            t = lax.rem(my_pos + o, N_DEV)
            pl.semaphore_signal(barrier_sem, inc=1, device_id=(t,),
                                device_id_type=pl.DeviceIdType.MESH)

        q_all = jnp.dot(x_ref[:, :], wq_ref[:, :],
                        preferred_element_type=jnp.float32)
        for b in range(B):
            ctx_parts = []
            for h in range(H_PER):
                q = q_all[b * SQ:(b + 1) * SQ, h * DH:(h + 1) * DH]
                k = k_ref[b, :, h, :]
                v = v_ref[b, :, h, :]
                s = lax.dot_general(
                    q, k, (((1,), (1,)), ((), ())),
                    preferred_element_type=jnp.float32) * 0.125
                m = jnp.max(s, axis=-1, keepdims=True)
                e = jnp.exp(s - m)
                w = e / jnp.sum(e, axis=-1, keepdims=True)
                ctx_parts.append(jnp.dot(w, v,
                                         preferred_element_type=jnp.float32))
            ctx_b = jnp.concatenate(ctx_parts, axis=1)
            p_b = jnp.dot(ctx_b, wo_ref[:, :],
                          preferred_element_type=jnp.float32)
            partial_ref[4 * b:4 * (b + 1)] = p_b.astype(jnp.bfloat16).reshape(
                4, ROWS_PER, D_MODEL)

        pl.semaphore_wait(barrier_sem, N_DEV - 1)

        drains = []
        own_a = []
        for c in range(NC):
            cols = pl.ds(c * CW, CW)
            own = pltpu.make_async_copy(
                partial_ref.at[my_pos, :, cols], comm_a.at[c, 0],
                local_sems.at[c])
            own.start()
            own_a.append(own)
            for o in range(1, N_DEV):
                t = lax.rem(my_pos + o, N_DEV)
                rdma = pltpu.make_async_remote_copy(
                    src_ref=partial_ref.at[t, :, cols],
                    dst_ref=comm_a.at[c, o],
                    send_sem=send_a.at[c, o],
                    recv_sem=recv_a.at[c, o],
                    device_id=(t,),
                    device_id_type=pl.DeviceIdType.MESH,
                )
                rdma.start()
                drains.append(rdma)
        for c in range(NC):
            own_a[c].wait()
            for o in range(1, N_DEV):
                pltpu.make_async_remote_copy(
                    src_ref=comm_a.at[c, o],
                    dst_ref=comm_a.at[c, o],
                    send_sem=send_a.at[c, o],
                    recv_sem=recv_a.at[c, o],
                    device_id=(my_pos,),
                    device_id_type=pl.DeviceIdType.MESH,
                ).wait_recv()
            red = comm_a[c, 0].astype(jnp.float32)
            for o in range(1, N_DEV):
                red = red + comm_a[c, o].astype(jnp.float32)
            comm_b[c] = red.astype(jnp.bfloat16)
            for o in range(1, N_DEV):
                t = lax.rem(my_pos + o, N_DEV)
                rdma = pltpu.make_async_remote_copy(
                    src_ref=comm_b.at[c],
                    dst_ref=gather_ref.at[c, my_pos],
                    send_sem=send_b.at[c, o],
                    recv_sem=recv_b.at[c, my_pos],
                    device_id=(t,),
                    device_id_type=pl.DeviceIdType.MESH,
                )
                rdma.start()
                drains.append(rdma)
            for p in range(N_DEV):
                @pl.when(p == my_pos)
                def _():
                    gather_ref[c, p] = comm_b[c]
        for c in range(NC):
            cols = pl.ds(c * CW, CW)
            for p in range(N_DEV):
                @pl.when(p != my_pos)
                def _():
                    pltpu.make_async_remote_copy(
                        src_ref=comm_b.at[c],
                        dst_ref=gather_ref.at[c, p],
                        send_sem=send_b.at[c, p],
                        recv_sem=recv_b.at[c, p],
                        device_id=(p,),
                        device_id_type=pl.DeviceIdType.MESH,
                    ).wait_recv()
            for p in range(N_DEV):
                out_ref[p // 4, 32 * (p % 4):32 * (p % 4) + 32, cols] = (
                    gather_ref[c, p].astype(jnp.float32))
        for rdma in drains:
            rdma.wait_send()

    out = pl.pallas_call(
        body,
        out_shape=jax.ShapeDtypeStruct((B, SQ, D_MODEL), jnp.float32),
        in_specs=[pl.BlockSpec(memory_space=pltpu.VMEM)] * 5,
        out_specs=pl.BlockSpec(memory_space=pltpu.VMEM),
        scratch_shapes=[
            pltpu.VMEM((N_DEV, ROWS_PER, D_MODEL), jnp.bfloat16),
            pltpu.VMEM((NC, N_DEV, ROWS_PER, CW), jnp.bfloat16),
            pltpu.VMEM((NC, ROWS_PER, CW), jnp.bfloat16),
            pltpu.VMEM((NC, N_DEV, ROWS_PER, CW), jnp.bfloat16),
            pltpu.SemaphoreType.DMA((NC,)),
            pltpu.SemaphoreType.DMA((NC, N_DEV)),
            pltpu.SemaphoreType.DMA((NC, N_DEV)),
            pltpu.SemaphoreType.DMA((NC, N_DEV)),
            pltpu.SemaphoreType.DMA((NC, N_DEV)),
        ],
        compiler_params=pltpu.CompilerParams(collective_id=0),
    )(xf, Wq, Ks, Vs, Wo)
    return out
